# baseline (speedup 1.0000x reference)
import functools

import jax
import jax.numpy as jnp
from jax import lax
from jax.experimental import pallas as pl
from jax.experimental.pallas import tpu as pltpu
from jax.experimental.pallas import tpu_sc as plsc

N = 10000
E = 320000
D = 128
NC, NS = 2, 16
HALF = 5120
N1 = N - HALF
NPH = 5248
RPT = NPH // NS
CH = 128
CPT = 88
EPT = CH * CPT
CAPC = EPT * NS
NBUF = 4
KOFF = 2
NSTEP = ((CPT + KOFF + NBUF - 1) // NBUF) * NBUF

_mesh = plsc.VectorSubcoreMesh(
    core_axis_name="c", subcore_axis_name="s", num_cores=NC, num_subcores=NS
)


def _deg_counts(dst2d, zeros16, ones16):
    @functools.partial(
        pl.kernel,
        out_type=jax.ShapeDtypeStruct((NC, NPH, 16), jnp.float32),
        mesh=_mesh,
        scratch_types=[
            pltpu.VMEM((CPT, CH), jnp.int32),
            pltpu.VMEM((CH, 16), jnp.float32),
            pltpu.VMEM_SHARED((NPH, 16), jnp.float32),
            pltpu.SemaphoreType.DMA,
        ],
    )
    def k(dst_hbm, z_hbm, ones_hbm, out_hbm, didx, ones_v, acc, sem):
        c = lax.axis_index("c")
        s = lax.axis_index("s")
        wid = c * NS + s
        pltpu.sync_copy(dst_hbm.at[pl.ds(wid * CPT, CPT)], didx)
        pltpu.sync_copy(ones_hbm, ones_v)
        pltpu.sync_copy(z_hbm, acc.at[pl.ds(s * RPT, RPT)])
        plsc.subcore_barrier()

        @pl.loop(0, CPT)
        def _(j):
            pltpu.async_copy(ones_v, acc.at[didx.at[j]], sem, add=True)

        @pl.loop(0, CPT)
        def _(j):
            pltpu.make_async_copy(ones_v, acc.at[didx.at[0]], sem).wait()

        plsc.subcore_barrier()
        pltpu.sync_copy(
            acc.at[pl.ds(s * RPT, RPT)], out_hbm.at[c, pl.ds(s * RPT, RPT)]
        )

    return k(dst2d, zeros16, ones16)


def _edge_scatter(g, src2d, dst2d, zeros128):
    @functools.partial(
        pl.kernel,
        out_type=jax.ShapeDtypeStruct((NC, NPH, D), jnp.float32),
        mesh=_mesh,
        scratch_types=[
            pltpu.VMEM((CPT, CH), jnp.int32),
            pltpu.VMEM((CPT, CH), jnp.int32),
        ]
        + [pltpu.VMEM((CH, D), jnp.float32) for _ in range(NBUF)]
        + [pltpu.VMEM_SHARED((NPH, D), jnp.float32)]
        + [pltpu.SemaphoreType.DMA for _ in range(2 * NBUF)],
    )
    def k(g_hbm, src_hbm, dst_hbm, z_hbm, out_hbm, sidx, didx, *rest):
        rows = rest[:NBUF]
        acc = rest[NBUF]
        gsem = rest[NBUF + 1 : NBUF + 1 + NBUF]
        ssem = rest[NBUF + 1 + NBUF :]
        c = lax.axis_index("c")
        s = lax.axis_index("s")
        wid = c * NS + s
        pltpu.sync_copy(src_hbm.at[pl.ds(wid * CPT, CPT)], sidx)
        pltpu.sync_copy(dst_hbm.at[pl.ds(wid * CPT, CPT)], didx)
        pltpu.sync_copy(z_hbm, acc.at[pl.ds(s * RPT, RPT)])
        plsc.subcore_barrier()

        @pl.loop(0, NSTEP // NBUF)
        def _(i):
            for b in range(NBUF):
                j = i * NBUF + b
                bp = (b - KOFF) % NBUF

                @pl.when(j < CPT)
                def _():
                    @pl.when(j >= NBUF)
                    def _():
                        pltpu.make_async_copy(
                            rows[b], acc.at[didx.at[j]], ssem[b]
                        ).wait()

                    pltpu.async_copy(g_hbm.at[sidx.at[j]], rows[b], gsem[b])

                jj = j - KOFF

                @pl.when((jj >= 0) & (jj < CPT))
                def _():
                    pltpu.make_async_copy(
                        g_hbm.at[sidx.at[jj]], rows[bp], gsem[bp]
                    ).wait()
                    pltpu.async_copy(
                        rows[bp], acc.at[didx.at[jj]], ssem[bp], add=True
                    )

        for b in range(NBUF):
            pltpu.make_async_copy(rows[b], acc.at[didx.at[0]], ssem[b]).wait()

        plsc.subcore_barrier()
        pltpu.sync_copy(
            acc.at[pl.ds(s * RPT, RPT)], out_hbm.at[c, pl.ds(s * RPT, RPT)]
        )

    return k(g, src2d, dst2d, zeros128)


def _matmul(x, w):
    def body(x_ref, w_ref, o_ref):
        o_ref[...] = jnp.dot(
            x_ref[...], w_ref[...],
            precision=lax.Precision.HIGHEST,
            preferred_element_type=jnp.float32,
        )

    return pl.pallas_call(
        body, out_shape=jax.ShapeDtypeStruct((N, D), jnp.float32)
    )(x, w)


def _halves(S_ref):
    return jnp.concatenate([S_ref[0, :HALF, :], S_ref[1, :N1, :]], axis=0)


def _prep(counts, h):
    def body(c_ref, h_ref, g_ref, dinv_ref):
        deg = (
            jnp.concatenate(
                [c_ref[0, :HALF, 0:1], c_ref[1, :N1, 0:1]], axis=0
            )
            + 1.0
        )
        dinv = lax.rsqrt(deg)
        dinv_ref[...] = dinv
        g_ref[...] = h_ref[...] * dinv

    return pl.pallas_call(
        body,
        out_shape=(
            jax.ShapeDtypeStruct((N, D), jnp.float32),
            jax.ShapeDtypeStruct((N, 1), jnp.float32),
        ),
    )(counts, h)


def _mid(S, g, dinv, b, w):
    def body(S_ref, g_ref, dinv_ref, b_ref, w_ref, o_ref):
        x2 = dinv_ref[...] * (_halves(S_ref) + g_ref[...]) + b_ref[...]
        o_ref[...] = dinv_ref[...] * jnp.dot(
            x2, w_ref[...],
            precision=lax.Precision.HIGHEST,
            preferred_element_type=jnp.float32,
        )

    return pl.pallas_call(
        body, out_shape=jax.ShapeDtypeStruct((N, D), jnp.float32)
    )(S, g, dinv, b, w)


def _fin(S, g, dinv, b):
    def body(S_ref, g_ref, dinv_ref, b_ref, o_ref):
        o_ref[...] = dinv_ref[...] * (_halves(S_ref) + g_ref[...]) + b_ref[...]

    return pl.pallas_call(
        body, out_shape=jax.ShapeDtypeStruct((N, D), jnp.float32)
    )(S, g, dinv, b)


def kernel(x, edge_index, W1, b1, W2, b2, W3, b3):
    ei = edge_index.astype(jnp.int32)
    src, dst = ei[0], ei[1]

    hi = (dst >= HALF).astype(jnp.int32)
    lo = 1 - hi
    pos = jnp.where(
        hi == 0, jnp.cumsum(lo) - lo, CAPC + jnp.cumsum(hi) - hi
    )
    ldst = dst - hi * HALF
    src2d = (
        jnp.zeros((2 * CAPC,), jnp.int32)
        .at[pos].set(src, mode="drop")
        .reshape(2 * CAPC // CH, CH)
    )
    dst2d = (
        jnp.full((2 * CAPC,), HALF, jnp.int32)
        .at[pos].set(ldst, mode="drop")
        .reshape(2 * CAPC // CH, CH)
    )
    zeros128 = jnp.zeros((RPT, D), jnp.float32)
    zeros16 = jnp.zeros((RPT, 16), jnp.float32)
    ones16 = jnp.zeros((CH, 16), jnp.float32).at[:, 0].set(1.0)

    counts = _deg_counts(dst2d, zeros16, ones16)
    h1 = _matmul(x, W1)
    g1, dinv = _prep(counts, h1)
    S1 = _edge_scatter(g1, src2d, dst2d, zeros128)
    g2 = _mid(S1, g1, dinv, b1.reshape(1, D), W2)
    S2 = _edge_scatter(g2, src2d, dst2d, zeros128)
    g3 = _mid(S2, g2, dinv, b2.reshape(1, D), W3)
    S3 = _edge_scatter(g3, src2d, dst2d, zeros128)
    return _fin(S3, g3, dinv, b3.reshape(1, D))


# speedup vs baseline: 1.0063x; 1.0063x over previous
import functools

import jax
import jax.numpy as jnp
from jax import lax
from jax.experimental import pallas as pl
from jax.experimental.pallas import tpu as pltpu
from jax.experimental.pallas import tpu_sc as plsc

N = 10000
E = 320000
D = 128
NC, NS = 2, 16
HALF = 5120
N1 = N - HALF
NPH = 5248
RPT = NPH // NS
CH = 128
CPT = 88
EPT = CH * CPT
CAPC = EPT * NS
NBUF = 4
KOFF = 2
NSTEP = ((CPT + KOFF + NBUF - 1) // NBUF) * NBUF

_mesh = plsc.VectorSubcoreMesh(
    core_axis_name="c", subcore_axis_name="s", num_cores=NC, num_subcores=NS
)


def _deg_counts(dst2d, zeros16, ones16):
    @functools.partial(
        pl.kernel,
        out_type=jax.ShapeDtypeStruct((NC, NPH, 16), jnp.float32),
        mesh=_mesh,
        scratch_types=[
            pltpu.VMEM((CPT, CH), jnp.int32),
            pltpu.VMEM((CH, 16), jnp.float32),
            pltpu.VMEM_SHARED((NPH, 16), jnp.float32),
            pltpu.SemaphoreType.DMA,
        ],
    )
    def k(dst_hbm, z_hbm, ones_hbm, out_hbm, didx, ones_v, acc, sem):
        c = lax.axis_index("c")
        s = lax.axis_index("s")
        wid = c * NS + s
        pltpu.sync_copy(dst_hbm.at[pl.ds(wid * CPT, CPT)], didx)
        pltpu.sync_copy(ones_hbm, ones_v)
        pltpu.sync_copy(z_hbm, acc.at[pl.ds(s * RPT, RPT)])
        plsc.subcore_barrier()

        @pl.loop(0, CPT)
        def _(j):
            pltpu.async_copy(ones_v, acc.at[didx.at[j]], sem, add=True)

        @pl.loop(0, CPT)
        def _(j):
            pltpu.make_async_copy(ones_v, acc.at[didx.at[0]], sem).wait()

        plsc.subcore_barrier()
        pltpu.sync_copy(
            acc.at[pl.ds(s * RPT, RPT)], out_hbm.at[c, pl.ds(s * RPT, RPT)]
        )

    return k(dst2d, zeros16, ones16)


def _edge_scatter(g, src2d, dst2d, zeros128):
    @functools.partial(
        pl.kernel,
        out_type=jax.ShapeDtypeStruct((NC, NPH, D), jnp.float32),
        mesh=_mesh,
        scratch_types=[
            pltpu.VMEM((CPT, CH), jnp.int32),
            pltpu.VMEM((CPT, CH), jnp.int32),
        ]
        + [pltpu.VMEM((CH, D), jnp.float32) for _ in range(NBUF)]
        + [pltpu.VMEM_SHARED((NPH, D), jnp.float32)]
        + [pltpu.SemaphoreType.DMA for _ in range(2 * NBUF)],
    )
    def k(g_hbm, src_hbm, dst_hbm, z_hbm, out_hbm, sidx, didx, *rest):
        rows = rest[:NBUF]
        acc = rest[NBUF]
        gsem = rest[NBUF + 1 : NBUF + 1 + NBUF]
        ssem = rest[NBUF + 1 + NBUF :]
        c = lax.axis_index("c")
        s = lax.axis_index("s")
        wid = c * NS + s
        pltpu.sync_copy(src_hbm.at[pl.ds(wid * CPT, CPT)], sidx)
        pltpu.sync_copy(dst_hbm.at[pl.ds(wid * CPT, CPT)], didx)
        pltpu.sync_copy(z_hbm, acc.at[pl.ds(s * RPT, RPT)])
        plsc.subcore_barrier()

        @pl.loop(0, CPT)
        def _(j):
            pltpu.sync_copy(g_hbm.at[sidx.at[j]], rows[0])
            pltpu.sync_copy(rows[0], acc.at[didx.at[j]], add=True)

        plsc.subcore_barrier()
        pltpu.sync_copy(
            acc.at[pl.ds(s * RPT, RPT)], out_hbm.at[c, pl.ds(s * RPT, RPT)]
        )

    return k(g, src2d, dst2d, zeros128)


def _matmul(x, w):
    def body(x_ref, w_ref, o_ref):
        o_ref[...] = jnp.dot(
            x_ref[...], w_ref[...],
            precision=lax.Precision.HIGHEST,
            preferred_element_type=jnp.float32,
        )

    return pl.pallas_call(
        body, out_shape=jax.ShapeDtypeStruct((N, D), jnp.float32)
    )(x, w)


def _halves(S_ref):
    return jnp.concatenate([S_ref[0, :HALF, :], S_ref[1, :N1, :]], axis=0)


def _prep(counts, h):
    def body(c_ref, h_ref, g_ref, dinv_ref):
        deg = (
            jnp.concatenate(
                [c_ref[0, :HALF, 0:1], c_ref[1, :N1, 0:1]], axis=0
            )
            + 1.0
        )
        dinv = lax.rsqrt(deg)
        dinv_ref[...] = dinv
        g_ref[...] = h_ref[...] * dinv

    return pl.pallas_call(
        body,
        out_shape=(
            jax.ShapeDtypeStruct((N, D), jnp.float32),
            jax.ShapeDtypeStruct((N, 1), jnp.float32),
        ),
    )(counts, h)


def _mid(S, g, dinv, b, w):
    def body(S_ref, g_ref, dinv_ref, b_ref, w_ref, o_ref):
        x2 = dinv_ref[...] * (_halves(S_ref) + g_ref[...]) + b_ref[...]
        o_ref[...] = dinv_ref[...] * jnp.dot(
            x2, w_ref[...],
            precision=lax.Precision.HIGHEST,
            preferred_element_type=jnp.float32,
        )

    return pl.pallas_call(
        body, out_shape=jax.ShapeDtypeStruct((N, D), jnp.float32)
    )(S, g, dinv, b, w)


def _fin(S, g, dinv, b):
    def body(S_ref, g_ref, dinv_ref, b_ref, o_ref):
        o_ref[...] = dinv_ref[...] * (_halves(S_ref) + g_ref[...]) + b_ref[...]

    return pl.pallas_call(
        body, out_shape=jax.ShapeDtypeStruct((N, D), jnp.float32)
    )(S, g, dinv, b)


def kernel(x, edge_index, W1, b1, W2, b2, W3, b3):
    ei = edge_index.astype(jnp.int32)
    src, dst = ei[0], ei[1]

    hi = (dst >= HALF).astype(jnp.int32)
    lo = 1 - hi
    pos = jnp.where(
        hi == 0, jnp.cumsum(lo) - lo, CAPC + jnp.cumsum(hi) - hi
    )
    ldst = dst - hi * HALF
    src2d = (
        jnp.zeros((2 * CAPC,), jnp.int32)
        .at[pos].set(src, mode="drop")
        .reshape(2 * CAPC // CH, CH)
    )
    dst2d = (
        jnp.full((2 * CAPC,), HALF, jnp.int32)
        .at[pos].set(ldst, mode="drop")
        .reshape(2 * CAPC // CH, CH)
    )
    zeros128 = jnp.zeros((RPT, D), jnp.float32)
    zeros16 = jnp.zeros((RPT, 16), jnp.float32)
    ones16 = jnp.zeros((CH, 16), jnp.float32).at[:, 0].set(1.0)

    counts = _deg_counts(dst2d, zeros16, ones16)
    h1 = _matmul(x, W1)
    g1, dinv = _prep(counts, h1)
    S1 = _edge_scatter(g1, src2d, dst2d, zeros128)
    g2 = _mid(S1, g1, dinv, b1.reshape(1, D), W2)
    S2 = _edge_scatter(g2, src2d, dst2d, zeros128)
    g3 = _mid(S2, g2, dinv, b2.reshape(1, D), W3)
    S3 = _edge_scatter(g3, src2d, dst2d, zeros128)
    return _fin(S3, g3, dinv, b3.reshape(1, D))


# speedup vs baseline: 17.4812x; 17.3724x over previous
import functools

import jax
import jax.numpy as jnp
from jax import lax
from jax.experimental import pallas as pl
from jax.experimental.pallas import tpu as pltpu
from jax.experimental.pallas import tpu_sc as plsc

N = 10000
E = 320000
D = 128
NC, NS = 2, 16
CH = 128
CPT = 80
EPT = CH * CPT
EP = EPT * NC * NS
NCH = EP // CH
NP = 10112
RPT = NP // NS
NJ = NP - N
NBUF = 3

_mesh = plsc.VectorSubcoreMesh(
    core_axis_name="c", subcore_axis_name="s", num_cores=NC, num_subcores=NS
)


def _deg_counts(dst2d, zeros16, ones16):
    @functools.partial(
        pl.kernel,
        out_type=jax.ShapeDtypeStruct((NC, NP, 16), jnp.float32),
        mesh=_mesh,
        scratch_types=[
            pltpu.VMEM((CPT, CH), jnp.int32),
            pltpu.VMEM((CH, 16), jnp.float32),
            pltpu.VMEM_SHARED((NP, 16), jnp.float32),
        ],
    )
    def k(dst_hbm, z_hbm, ones_hbm, out_hbm, didx, ones_v, acc):
        c = lax.axis_index("c")
        s = lax.axis_index("s")
        wid = c * NS + s
        pltpu.sync_copy(dst_hbm.at[pl.ds(wid * CPT, CPT)], didx)
        pltpu.sync_copy(ones_hbm, ones_v)
        pltpu.sync_copy(z_hbm, acc.at[pl.ds(s * RPT, RPT)])
        plsc.subcore_barrier()

        @pl.loop(0, CPT)
        def _(j):
            pltpu.sync_copy(ones_v, acc.at[didx.at[j]], add=True)

        plsc.subcore_barrier()
        pltpu.sync_copy(
            acc.at[pl.ds(s * RPT, RPT)], out_hbm.at[c, pl.ds(s * RPT, RPT)]
        )

    return k(dst2d, zeros16, ones16)


def _edge_scatter(g, idx3d, zeros128):
    @functools.partial(
        pl.kernel,
        out_type=jax.ShapeDtypeStruct((NC, NP, D), jnp.float32),
        mesh=_mesh,
        scratch_types=[pltpu.VMEM((NBUF, 2, CH), jnp.int32)]
        + [pltpu.VMEM((CH, D), jnp.float32) for _ in range(NBUF)]
        + [pltpu.VMEM_SHARED((NP, D), jnp.float32)]
        + [pltpu.SemaphoreType.DMA for _ in range(3 * NBUF)],
    )
    def k(g_hbm, idx_hbm, z_hbm, out_hbm, iring, *rest):
        rows = rest[:NBUF]
        acc = rest[NBUF]
        isem = rest[NBUF + 1 : NBUF + 1 + NBUF]
        gsem = rest[NBUF + 1 + NBUF : NBUF + 1 + 2 * NBUF]
        ssem = rest[NBUF + 1 + 2 * NBUF :]
        c = lax.axis_index("c")
        s = lax.axis_index("s")
        wid = c * NS + s
        base = wid * CPT
        pltpu.sync_copy(z_hbm, acc.at[pl.ds(s * RPT, RPT)])
        plsc.subcore_barrier()

        nstep = CPT + 2
        nouter = (nstep + NBUF - 1) // NBUF

        @pl.loop(0, nouter)
        def _(i):
            for b in range(NBUF):
                t = i * NBUF + b
                sl_g = (b - 1) % NBUF
                sl_s = (b - 2) % NBUF

                @pl.when(t < CPT)
                def _():
                    pltpu.async_copy(
                        idx_hbm.at[base + t], iring.at[b], isem[b]
                    )

                cg = t - 1

                @pl.when((cg >= 0) & (cg < CPT))
                def _():
                    @pl.when(cg >= NBUF)
                    def _():
                        pltpu.make_async_copy(
                            rows[sl_g],
                            acc.at[iring.at[sl_g, 1]],
                            ssem[sl_g],
                        ).wait()

                    pltpu.make_async_copy(
                        idx_hbm.at[base], iring.at[sl_g], isem[sl_g]
                    ).wait()
                    pltpu.async_copy(
                        g_hbm.at[iring.at[sl_g, 0]], rows[sl_g], gsem[sl_g]
                    )

                cs = t - 2

                @pl.when((cs >= 0) & (cs < CPT))
                def _():
                    pltpu.make_async_copy(
                        g_hbm.at[iring.at[sl_s, 0]], rows[sl_s], gsem[sl_s]
                    ).wait()
                    pltpu.async_copy(
                        rows[sl_s], acc.at[iring.at[sl_s, 1]], ssem[sl_s],
                        add=True,
                    )

        for b in range(NBUF):
            pltpu.make_async_copy(
                rows[b], acc.at[iring.at[b, 1]], ssem[b]
            ).wait()

        plsc.subcore_barrier()
        pltpu.sync_copy(
            acc.at[pl.ds(s * RPT, RPT)], out_hbm.at[c, pl.ds(s * RPT, RPT)]
        )

    return k(g, idx3d, zeros128)


def _matmul(x, w):
    def body(x_ref, w_ref, o_ref):
        o_ref[...] = jnp.dot(
            x_ref[...], w_ref[...],
            precision=lax.Precision.HIGHEST,
            preferred_element_type=jnp.float32,
        )

    return pl.pallas_call(
        body, out_shape=jax.ShapeDtypeStruct((N, D), jnp.float32)
    )(x, w)


def _prep(counts, h):
    def body(c_ref, h_ref, g_ref, dinv_ref):
        deg = c_ref[0, :N, 0:1] + c_ref[1, :N, 0:1] + 1.0
        dinv = lax.rsqrt(deg)
        dinv_ref[...] = dinv
        g_ref[...] = h_ref[...] * dinv

    return pl.pallas_call(
        body,
        out_shape=(
            jax.ShapeDtypeStruct((N, D), jnp.float32),
            jax.ShapeDtypeStruct((N, 1), jnp.float32),
        ),
    )(counts, h)


def _mid(S, g, dinv, b, w):
    def body(S_ref, g_ref, dinv_ref, b_ref, w_ref, o_ref):
        sm = S_ref[0, :N, :] + S_ref[1, :N, :]
        x2 = dinv_ref[...] * (sm + g_ref[...]) + b_ref[...]
        o_ref[...] = dinv_ref[...] * jnp.dot(
            x2, w_ref[...],
            precision=lax.Precision.HIGHEST,
            preferred_element_type=jnp.float32,
        )

    return pl.pallas_call(
        body, out_shape=jax.ShapeDtypeStruct((N, D), jnp.float32)
    )(S, g, dinv, b, w)


def _fin(S, g, dinv, b):
    def body(S_ref, g_ref, dinv_ref, b_ref, o_ref):
        sm = S_ref[0, :N, :] + S_ref[1, :N, :]
        o_ref[...] = dinv_ref[...] * (sm + g_ref[...]) + b_ref[...]

    return pl.pallas_call(
        body, out_shape=jax.ShapeDtypeStruct((N, D), jnp.float32)
    )(S, g, dinv, b)


def kernel(x, edge_index, W1, b1, W2, b2, W3, b3):
    ei = edge_index.astype(jnp.int32)
    pad = EP - E
    pad_src = jnp.arange(pad, dtype=jnp.int32) % N
    pad_dst = N + (jnp.arange(pad, dtype=jnp.int32) % NJ)
    src2d = jnp.concatenate([ei[0], pad_src]).reshape(NCH, CH)
    dst2d = jnp.concatenate([ei[1], pad_dst]).reshape(NCH, CH)
    idx3d = jnp.stack([src2d, dst2d], axis=1)
    zeros128 = jnp.zeros((RPT, D), jnp.float32)
    zeros16 = jnp.zeros((RPT, 16), jnp.float32)
    ones16 = jnp.zeros((CH, 16), jnp.float32).at[:, 0].set(1.0)

    counts = _deg_counts(dst2d, zeros16, ones16)
    h1 = _matmul(x, W1)
    g1, dinv = _prep(counts, h1)
    S1 = _edge_scatter(g1, idx3d, zeros128)
    g2 = _mid(S1, g1, dinv, b1.reshape(1, D), W2)
    S2 = _edge_scatter(g2, idx3d, zeros128)
    g3 = _mid(S2, g2, dinv, b2.reshape(1, D), W3)
    S3 = _edge_scatter(g3, idx3d, zeros128)
    return _fin(S3, g3, dinv, b3.reshape(1, D))
